# baseline (speedup 1.0000x reference)
import jax
import jax.numpy as jnp
from jax.experimental import pallas as pl

NUM_CODES = 1024
HIDDEN = 256
BETA = 0.25


def _vq_body(z_ref, w_ref, zq_ref, idx_ref, loss_ref):
    i = pl.program_id(0)
    nsteps = pl.num_programs(0)

    w = w_ref[...]
    w2 = jnp.sum(w * w, axis=1)
    zb = z_ref[0]
    zf2 = jnp.sum(zb * zb, axis=0, keepdims=True)

    s = jax.lax.dot_general(
        w, zb, (((1,), (0,)), ((), ())),
        preferred_element_type=jnp.float32)
    d = (zf2 + w2[:, None]) - 2.0 * s

    dmin = jnp.min(d, axis=0, keepdims=True)
    row = jax.lax.broadcasted_iota(jnp.int32, d.shape, 0)
    idx = jnp.min(jnp.where(d == dmin, row, NUM_CODES), axis=0)
    idx_ref[...] = idx.reshape(idx_ref.shape)

    onehot = (idx[None, :] == row).astype(jnp.float32)
    zq = jax.lax.dot_general(
        w, onehot, (((0,), (0,)), ((), ())),
        preferred_element_type=jnp.float32,
        precision=jax.lax.Precision.HIGHEST)
    zq_ref[0] = zb + (zq - zb)

    part = jnp.sum(dmin).reshape(1, 1)

    @pl.when(i == 0)
    def _init():
        loss_ref[...] = jnp.zeros_like(loss_ref)

    loss_ref[...] += part

    @pl.when(i == nsteps - 1)
    def _final():
        n_elems = nsteps * zb.shape[1] * HIDDEN
        loss_ref[...] = loss_ref[...] * ((1.0 + BETA) / n_elems)


def kernel(z, W):
    B, C, H, Wsp = z.shape
    P = H * Wsp
    zr = z.reshape(B, C, P)

    zq, idx, loss = pl.pallas_call(
        _vq_body,
        grid=(B,),
        in_specs=[
            pl.BlockSpec((1, C, P), lambda i: (i, 0, 0)),
            pl.BlockSpec((NUM_CODES, C), lambda i: (0, 0)),
        ],
        out_specs=[
            pl.BlockSpec((1, C, P), lambda i: (i, 0, 0)),
            pl.BlockSpec((1, 1, P), lambda i: (i, 0, 0)),
            pl.BlockSpec((1, 1), lambda i: (0, 0)),
        ],
        out_shape=[
            jax.ShapeDtypeStruct((B, C, P), jnp.float32),
            jax.ShapeDtypeStruct((B, 1, P), jnp.int32),
            jax.ShapeDtypeStruct((1, 1), jnp.float32),
        ],
    )(zr, W)

    return (zq.reshape(B, C, H, Wsp), idx.reshape(B * P), loss[0, 0])


# speedup vs baseline: 1.6819x; 1.6819x over previous
import jax
import jax.numpy as jnp
from jax.experimental import pallas as pl

NUM_CODES = 1024
HIDDEN = 256
BETA = 0.25


def _vq_body(z_ref, w_ref, zq_ref, idx_ref, loss_ref):
    i = pl.program_id(0)
    nsteps = pl.num_programs(0)

    w = w_ref[...]
    w2 = jnp.sum(w * w, axis=1)
    zb = z_ref[0]
    zf2 = jnp.sum(zb * zb, axis=0, keepdims=True)

    s = jax.lax.dot_general(
        w, zb, (((1,), (0,)), ((), ())),
        preferred_element_type=jnp.float32)
    d = (zf2 + w2[:, None]) - 2.0 * s

    dmin = jnp.min(d, axis=0, keepdims=True)
    idx = jnp.argmin(d, axis=0).astype(jnp.int32)
    idx_ref[...] = idx.reshape(idx_ref.shape)

    row = jax.lax.broadcasted_iota(jnp.int32, d.shape, 0)
    onehot = (idx[None, :] == row).astype(jnp.float32)
    zq = jax.lax.dot_general(
        w, onehot, (((0,), (0,)), ((), ())),
        preferred_element_type=jnp.float32)
    zq_ref[0] = zb + (zq - zb)

    part = jnp.sum(dmin).reshape(1, 1)

    @pl.when(i == 0)
    def _init():
        loss_ref[...] = jnp.zeros_like(loss_ref)

    loss_ref[...] += part

    @pl.when(i == nsteps - 1)
    def _final():
        n_elems = nsteps * zb.shape[1] * HIDDEN
        loss_ref[...] = loss_ref[...] * ((1.0 + BETA) / n_elems)


def kernel(z, W):
    B, C, H, Wsp = z.shape
    P = H * Wsp
    zr = z.reshape(B, C, P)

    zq, idx, loss = pl.pallas_call(
        _vq_body,
        grid=(B,),
        in_specs=[
            pl.BlockSpec((1, C, P), lambda i: (i, 0, 0)),
            pl.BlockSpec((NUM_CODES, C), lambda i: (0, 0)),
        ],
        out_specs=[
            pl.BlockSpec((1, C, P), lambda i: (i, 0, 0)),
            pl.BlockSpec((1, 1, P), lambda i: (i, 0, 0)),
            pl.BlockSpec((1, 1), lambda i: (0, 0)),
        ],
        out_shape=[
            jax.ShapeDtypeStruct((B, C, P), jnp.float32),
            jax.ShapeDtypeStruct((B, 1, P), jnp.int32),
            jax.ShapeDtypeStruct((1, 1), jnp.float32),
        ],
    )(zr, W)

    return (zq.reshape(B, C, H, Wsp), idx.reshape(B * P), loss[0, 0])
